# Initial kernel scaffold; baseline (speedup 1.0000x reference)
import jax
import jax.numpy as jnp
from jax.experimental import pallas as pl


def kernel(f, x_e, src, Wq, bq, Wk, bk, Wv, bv, Wo, bo, W0, b0, W1, b1, Wl, bl):
    raise NotImplementedError("write your pallas kernel here")


# speedup vs baseline: 1.5193x; 1.5193x over previous
import functools
import math

import jax
import jax.numpy as jnp
from jax import lax
from jax.experimental import pallas as pl
from jax.experimental.pallas import tpu as pltpu
from jax.experimental.pallas import tpu_sc as plsc

N = 10000
K = 32
E = N * K
D = 128
H = 4
DK = D // H

BN = 200
BE = BN * K
GRID = N // BN

_INV_SQRT_DK = 1.0 / math.sqrt(DK)


def _dot(a, b):
    return jnp.dot(a, b, preferred_element_type=jnp.float32,
                   precision=lax.Precision.HIGHEST)


def _pairswap(x):
    r = x.shape[0]
    x3 = x.reshape(r // 2, 2, D)
    sw = jnp.concatenate([x3[:, 1:2, :], x3[:, 0:1, :]], axis=1)
    return sw.reshape(r, D)


def _mha_residual(fh, mail, Wq, bq, Wk, bk, Wv, bv, Wo, bo):
    q = _dot(fh, Wq) + bq
    k = _dot(mail, Wk) + bk
    v = _dot(mail, Wv) + bv
    outs = []
    for h in range(H):
        sl = slice(h * DK, (h + 1) * DK)
        qh = q[:, sl]
        kh = k[:, sl].reshape(BN, K, DK)
        vh = v[:, sl].reshape(BN, K, DK)
        s = (qh[:, None, :] * kh).sum(axis=-1) * _INV_SQRT_DK
        s = s - s.max(axis=1, keepdims=True)
        p = jnp.exp(s)
        p = p / p.sum(axis=1, keepdims=True)
        outs.append((p[:, :, None] * vh).sum(axis=1))
    o = jnp.concatenate(outs, axis=1)
    return _dot(o, Wo) + bo + fh


def _p1_body(xe_ref, f_ref, Wq_ref, bq_ref, Wk_ref, bk_ref, Wv_ref, bv_ref,
             Wo_ref, bo_ref, out_ref):
    out_ref[...] = _mha_residual(
        f_ref[...], xe_ref[...],
        Wq_ref[...], bq_ref[...], Wk_ref[...], bk_ref[...],
        Wv_ref[...], bv_ref[...], Wo_ref[...], bo_ref[...])


def _p2_body(xe_ref, g_ref, fh1_ref, Wq_ref, bq_ref, Wk_ref, bk_ref,
             Wv_ref, bv_ref, Wo_ref, bo_ref, W0_ref, b0_ref,
             h1_ref, fh2_ref):
    xe = xe_ref[...]
    m = g_ref[...] - _pairswap(xe)
    h1 = jnp.maximum(xe + _dot(m, W0_ref[...]) + b0_ref[...], 0.0)
    h1_ref[...] = h1
    fh2_ref[...] = _mha_residual(
        fh1_ref[...], h1,
        Wq_ref[...], bq_ref[...], Wk_ref[...], bk_ref[...],
        Wv_ref[...], bv_ref[...], Wo_ref[...], bo_ref[...])


def _p3_body(xe_ref, g_ref, h1_ref, fh2_ref, f_ref, W1_ref, b1_ref,
             Wl_ref, bl_ref, out_ref):
    xe = xe_ref[...]
    m = g_ref[...] - _pairswap(h1_ref[...])
    h2 = jnp.maximum(xe + _dot(m, W1_ref[...]) + b1_ref[...], 0.0)
    mail_sum = h2.reshape(BN, K, D).sum(axis=1)
    Wl = Wl_ref[...]
    out_ref[...] = (_dot(mail_sum, Wl[0:D]) + _dot(fh2_ref[...], Wl[D:2 * D])
                    + _dot(f_ref[...], Wl[2 * D:3 * D]) + bl_ref[...])


def _edge_spec():
    return pl.BlockSpec((BE, D), lambda i: (i, 0))


def _node_spec():
    return pl.BlockSpec((BN, D), lambda i: (i, 0))


def _w_spec(rows):
    return pl.BlockSpec((rows, D), lambda i: (0, 0))


def _b_spec():
    return pl.BlockSpec((1, D), lambda i: (0, 0))


def _make_sc_gather():
    info = plsc.get_sparse_core_info()
    nw = info.num_cores * info.num_subcores
    per_w = E // nw
    ch = 200
    n_pairs = per_w // (2 * ch)
    mesh = plsc.VectorSubcoreMesh(core_axis_name="c", subcore_axis_name="s")

    @functools.partial(
        pl.kernel,
        out_type=jax.ShapeDtypeStruct((E, D), jnp.float32),
        mesh=mesh,
        scratch_types=[
            pltpu.VMEM((ch,), jnp.int32),
            pltpu.VMEM((ch,), jnp.int32),
            pltpu.VMEM((ch, D), jnp.float32),
            pltpu.VMEM((ch, D), jnp.float32),
            pltpu.SemaphoreType.DMA,
            pltpu.SemaphoreType.DMA,
        ],
    )
    def gather(table_hbm, idx_hbm, out_hbm, idx_a, idx_b, rows_a, rows_b,
               sem_a, sem_b):
        wid = lax.axis_index("s") * info.num_cores + lax.axis_index("c")
        base = wid * per_w
        idx_v = (idx_a, idx_b)
        rows_v = (rows_a, rows_b)
        sems = (sem_a, sem_b)

        def body(i, _):
            handles = []
            for b in range(2):
                off = base + (2 * i + b) * ch
                pltpu.sync_copy(idx_hbm.at[pl.ds(off, ch)], idx_v[b])
                handles.append(
                    pltpu.async_copy(table_hbm.at[idx_v[b]], rows_v[b],
                                     sems[b]))
            for b in range(2):
                off = base + (2 * i + b) * ch
                handles[b].wait()
                pltpu.sync_copy(rows_v[b], out_hbm.at[pl.ds(off, ch)])
            return ()

        lax.fori_loop(0, n_pairs, body, ())

    return gather


def kernel(f, x_e, src, Wq, bq, Wk, bk, Wv, bv, Wo, bo, W0, b0, W1, b1,
           Wl, bl):
    bq2, bk2, bv2, bo2, b02, b12, bl2 = (
        b.reshape(1, D) for b in (bq, bk, bv, bo, b0, b1, bl))

    p1 = pl.pallas_call(
        _p1_body,
        grid=(GRID,),
        in_specs=[_edge_spec(), _node_spec(),
                  _w_spec(D), _b_spec(), _w_spec(D), _b_spec(),
                  _w_spec(D), _b_spec(), _w_spec(D), _b_spec()],
        out_specs=_node_spec(),
        out_shape=jax.ShapeDtypeStruct((N, D), jnp.float32),
    )
    fh1 = p1(x_e, f, Wq, bq2, Wk, bk2, Wv, bv2, Wo, bo2)

    sc_gather = _make_sc_gather()
    g0 = sc_gather(fh1, src)

    p2 = pl.pallas_call(
        _p2_body,
        grid=(GRID,),
        in_specs=[_edge_spec(), _edge_spec(), _node_spec(),
                  _w_spec(D), _b_spec(), _w_spec(D), _b_spec(),
                  _w_spec(D), _b_spec(), _w_spec(D), _b_spec(),
                  _w_spec(D), _b_spec()],
        out_specs=[_edge_spec(), _node_spec()],
        out_shape=[jax.ShapeDtypeStruct((E, D), jnp.float32),
                   jax.ShapeDtypeStruct((N, D), jnp.float32)],
    )
    h1, fh2 = p2(x_e, g0, fh1, Wq, bq2, Wk, bk2, Wv, bv2, Wo, bo2, W0, b02)

    g1 = sc_gather(fh2, src)

    p3 = pl.pallas_call(
        _p3_body,
        grid=(GRID,),
        in_specs=[_edge_spec(), _edge_spec(), _edge_spec(), _node_spec(),
                  _node_spec(), _w_spec(D), _b_spec(),
                  pl.BlockSpec((3 * D, D), lambda i: (0, 0)), _b_spec()],
        out_specs=_node_spec(),
        out_shape=jax.ShapeDtypeStruct((N, D), jnp.float32),
    )
    return p3(x_e, g1, h1, fh2, f, W1, b12, Wl, bl2)


# speedup vs baseline: 3.5964x; 2.3671x over previous
import functools
import math

import jax
import jax.numpy as jnp
from jax import lax
from jax.experimental import pallas as pl
from jax.experimental.pallas import tpu as pltpu
from jax.experimental.pallas import tpu_sc as plsc

N = 10000
K = 32
E = N * K
D = 128
H = 4
DK = D // H

BN = 200
BE = BN * K
GRID = N // BN

_INV_SQRT_DK = 1.0 / math.sqrt(DK)


def _dot(a, b):
    return jnp.dot(a, b, preferred_element_type=jnp.float32,
                   precision=lax.Precision.HIGHEST)


def _pairswap(x):
    r = x.shape[0]
    up = jnp.roll(x, -1, axis=0)
    dn = jnp.roll(x, 1, axis=0)
    row = lax.broadcasted_iota(jnp.int32, (r, D), 0)
    return jnp.where(row % 2 == 0, up, dn)


def _head_blockdiag():
    d = lax.broadcasted_iota(jnp.int32, (D, D), 0)
    c = lax.broadcasted_iota(jnp.int32, (D, D), 1)
    return jnp.where(d // DK == c // K, 1.0, 0.0).astype(jnp.float32)


def _segsum_k(x):
    return x.reshape(x.shape[0] // K, K, D).sum(axis=1)


def _mha_residual(fh, mail, Wq, bq, Wk, bk, Wv, bv, Wo, bo):
    q = _dot(fh, Wq) + bq
    k = _dot(mail, Wk) + bk
    v = _dot(mail, Wv) + bv
    qe = jnp.broadcast_to(q[:, None, :], (BN, K, D)).reshape(BE, D)
    s = _dot(qe * k, _head_blockdiag()) * _INV_SQRT_DK
    u = jnp.exp(s)
    numer = _segsum_k(u * v)
    denom = _segsum_k(u)
    o = numer / denom
    return _dot(o, Wo) + bo + fh


def _p1_body(xe_ref, f_ref, Wq_ref, bq_ref, Wk_ref, bk_ref, Wv_ref, bv_ref,
             Wo_ref, bo_ref, out_ref):
    out_ref[...] = _mha_residual(
        f_ref[...], xe_ref[...],
        Wq_ref[...], bq_ref[...], Wk_ref[...], bk_ref[...],
        Wv_ref[...], bv_ref[...], Wo_ref[...], bo_ref[...])


def _p2_body(xe_ref, g_ref, fh1_ref, Wq_ref, bq_ref, Wk_ref, bk_ref,
             Wv_ref, bv_ref, Wo_ref, bo_ref, W0_ref, b0_ref,
             h1_ref, fh2_ref):
    xe = xe_ref[...]
    m = g_ref[...] - _pairswap(xe)
    h1 = jnp.maximum(xe + _dot(m, W0_ref[...]) + b0_ref[...], 0.0)
    h1_ref[...] = h1
    fh2_ref[...] = _mha_residual(
        fh1_ref[...], h1,
        Wq_ref[...], bq_ref[...], Wk_ref[...], bk_ref[...],
        Wv_ref[...], bv_ref[...], Wo_ref[...], bo_ref[...])


def _p3_body(xe_ref, g_ref, h1_ref, fh2_ref, f_ref, W1_ref, b1_ref,
             Wl_ref, bl_ref, out_ref):
    xe = xe_ref[...]
    m = g_ref[...] - _pairswap(h1_ref[...])
    h2 = jnp.maximum(xe + _dot(m, W1_ref[...]) + b1_ref[...], 0.0)
    mail_sum = h2.reshape(BN, K, D).sum(axis=1)
    Wl = Wl_ref[...]
    out_ref[...] = (_dot(mail_sum, Wl[0:D]) + _dot(fh2_ref[...], Wl[D:2 * D])
                    + _dot(f_ref[...], Wl[2 * D:3 * D]) + bl_ref[...])


def _edge_spec():
    return pl.BlockSpec((BE, D), lambda i: (i, 0))


def _node_spec():
    return pl.BlockSpec((BN, D), lambda i: (i, 0))


def _w_spec(rows):
    return pl.BlockSpec((rows, D), lambda i: (0, 0))


def _b_spec():
    return pl.BlockSpec((1, D), lambda i: (0, 0))


def _make_sc_gather():
    info = plsc.get_sparse_core_info()
    nw = info.num_cores * info.num_subcores
    per_w = E // nw
    ch = 200
    n_pairs = per_w // (2 * ch)
    mesh = plsc.VectorSubcoreMesh(core_axis_name="c", subcore_axis_name="s")

    @functools.partial(
        pl.kernel,
        out_type=jax.ShapeDtypeStruct((E, D), jnp.float32),
        mesh=mesh,
        scratch_types=[
            pltpu.VMEM((ch,), jnp.int32),
            pltpu.VMEM((ch,), jnp.int32),
            pltpu.VMEM((ch, D), jnp.float32),
            pltpu.VMEM((ch, D), jnp.float32),
            pltpu.SemaphoreType.DMA,
            pltpu.SemaphoreType.DMA,
        ],
    )
    def gather(table_hbm, idx_hbm, out_hbm, idx_a, idx_b, rows_a, rows_b,
               sem_a, sem_b):
        wid = lax.axis_index("s") * info.num_cores + lax.axis_index("c")
        base = wid * per_w
        idx_v = (idx_a, idx_b)
        rows_v = (rows_a, rows_b)
        sems = (sem_a, sem_b)

        def body(i, _):
            handles = []
            for b in range(2):
                off = base + (2 * i + b) * ch
                pltpu.sync_copy(idx_hbm.at[pl.ds(off, ch)], idx_v[b])
                handles.append(
                    pltpu.async_copy(table_hbm.at[idx_v[b]], rows_v[b],
                                     sems[b]))
            for b in range(2):
                off = base + (2 * i + b) * ch
                handles[b].wait()
                pltpu.sync_copy(rows_v[b], out_hbm.at[pl.ds(off, ch)])
            return ()

        lax.fori_loop(0, n_pairs, body, ())

    return gather


def kernel(f, x_e, src, Wq, bq, Wk, bk, Wv, bv, Wo, bo, W0, b0, W1, b1,
           Wl, bl):
    bq2, bk2, bv2, bo2, b02, b12, bl2 = (
        b.reshape(1, D) for b in (bq, bk, bv, bo, b0, b1, bl))

    p1 = pl.pallas_call(
        _p1_body,
        grid=(GRID,),
        in_specs=[_edge_spec(), _node_spec(),
                  _w_spec(D), _b_spec(), _w_spec(D), _b_spec(),
                  _w_spec(D), _b_spec(), _w_spec(D), _b_spec()],
        out_specs=_node_spec(),
        out_shape=jax.ShapeDtypeStruct((N, D), jnp.float32),
    )
    fh1 = p1(x_e, f, Wq, bq2, Wk, bk2, Wv, bv2, Wo, bo2)

    sc_gather = _make_sc_gather()
    g0 = sc_gather(fh1, src)

    p2 = pl.pallas_call(
        _p2_body,
        grid=(GRID,),
        in_specs=[_edge_spec(), _edge_spec(), _node_spec(),
                  _w_spec(D), _b_spec(), _w_spec(D), _b_spec(),
                  _w_spec(D), _b_spec(), _w_spec(D), _b_spec(),
                  _w_spec(D), _b_spec()],
        out_specs=[_edge_spec(), _node_spec()],
        out_shape=[jax.ShapeDtypeStruct((E, D), jnp.float32),
                   jax.ShapeDtypeStruct((N, D), jnp.float32)],
    )
    h1, fh2 = p2(x_e, g0, fh1, Wq, bq2, Wk, bk2, Wv, bv2, Wo, bo2, W0, b02)

    g1 = sc_gather(fh2, src)

    p3 = pl.pallas_call(
        _p3_body,
        grid=(GRID,),
        in_specs=[_edge_spec(), _edge_spec(), _edge_spec(), _node_spec(),
                  _node_spec(), _w_spec(D), _b_spec(),
                  pl.BlockSpec((3 * D, D), lambda i: (0, 0)), _b_spec()],
        out_specs=_node_spec(),
        out_shape=jax.ShapeDtypeStruct((N, D), jnp.float32),
    )
    return p3(x_e, g1, h1, fh2, f, W1, b12, Wl, bl2)


# speedup vs baseline: 9.1282x; 2.5381x over previous
import functools
import math

import jax
import jax.numpy as jnp
from jax import lax
from jax.experimental import pallas as pl
from jax.experimental.pallas import tpu as pltpu
from jax.experimental.pallas import tpu_sc as plsc

N = 10000
K = 32
E = N * K
D = 128
H = 4
DK = D // H

BN = 200
BE = BN * K
GRID = N // BN

_INV_SQRT_DK = 1.0 / math.sqrt(DK)


def _dot(a, b):
    return jnp.dot(a, b, preferred_element_type=jnp.float32,
                   precision=lax.Precision.DEFAULT)


def _pairswap(x):
    r = x.shape[0]
    up = jnp.roll(x, -1, axis=0)
    dn = jnp.roll(x, 1, axis=0)
    row = lax.broadcasted_iota(jnp.int32, (r, D), 0)
    return jnp.where(row % 2 == 0, up, dn)


def _head_blockdiag():
    d = lax.broadcasted_iota(jnp.int32, (D, D), 0)
    c = lax.broadcasted_iota(jnp.int32, (D, D), 1)
    return jnp.where(d // DK == c // K, 1.0, 0.0).astype(jnp.float32)


def _segsum_k(x):
    return x.reshape(x.shape[0] // K, K, D).sum(axis=1)


def _mha_residual(fh, mail, Wq, bq, Wk, bk, Wv, bv, Wo, bo):
    q = _dot(fh, Wq) + bq
    k = _dot(mail, Wk) + bk
    v = _dot(mail, Wv) + bv
    qe = jnp.broadcast_to(q[:, None, :], (BN, K, D)).reshape(BE, D)
    s = _dot(qe * k, _head_blockdiag()) * _INV_SQRT_DK
    u = jnp.exp(s)
    numer = _segsum_k(u * v)
    denom = _segsum_k(u)
    o = numer / denom
    return _dot(o, Wo) + bo + fh


def _p1_body(xe_ref, f_ref, Wq_ref, bq_ref, Wk_ref, bk_ref, Wv_ref, bv_ref,
             Wo_ref, bo_ref, out_ref):
    out_ref[...] = _mha_residual(
        f_ref[...], xe_ref[...],
        Wq_ref[...], bq_ref[...], Wk_ref[...], bk_ref[...],
        Wv_ref[...], bv_ref[...], Wo_ref[...], bo_ref[...])


def _p2_body(xe_ref, g_ref, fh1_ref, Wq_ref, bq_ref, Wk_ref, bk_ref,
             Wv_ref, bv_ref, Wo_ref, bo_ref, W0_ref, b0_ref,
             h1_ref, fh2_ref):
    xe = xe_ref[...]
    m = g_ref[...] - _pairswap(xe)
    h1 = jnp.maximum(xe + _dot(m, W0_ref[...]) + b0_ref[...], 0.0)
    h1_ref[...] = h1
    fh2_ref[...] = _mha_residual(
        fh1_ref[...], h1,
        Wq_ref[...], bq_ref[...], Wk_ref[...], bk_ref[...],
        Wv_ref[...], bv_ref[...], Wo_ref[...], bo_ref[...])


def _p3_body(xe_ref, g_ref, h1_ref, fh2_ref, f_ref, W1_ref, b1_ref,
             Wl_ref, bl_ref, out_ref):
    xe = xe_ref[...]
    m = g_ref[...] - _pairswap(h1_ref[...])
    h2 = jnp.maximum(xe + _dot(m, W1_ref[...]) + b1_ref[...], 0.0)
    mail_sum = h2.reshape(BN, K, D).sum(axis=1)
    Wl = Wl_ref[...]
    out_ref[...] = (_dot(mail_sum, Wl[0:D]) + _dot(fh2_ref[...], Wl[D:2 * D])
                    + _dot(f_ref[...], Wl[2 * D:3 * D]) + bl_ref[...])


def _edge_spec():
    return pl.BlockSpec((BE, D), lambda i: (i, 0))


def _node_spec():
    return pl.BlockSpec((BN, D), lambda i: (i, 0))


def _w_spec(rows):
    return pl.BlockSpec((rows, D), lambda i: (0, 0))


def _b_spec():
    return pl.BlockSpec((1, D), lambda i: (0, 0))


def _make_sc_gather():
    info = plsc.get_sparse_core_info()
    nw = info.num_cores * info.num_subcores
    per_w = E // nw
    ch = 200
    n_pairs = per_w // (2 * ch)
    mesh = plsc.VectorSubcoreMesh(core_axis_name="c", subcore_axis_name="s")

    @functools.partial(
        pl.kernel,
        out_type=jax.ShapeDtypeStruct((E, D), jnp.float32),
        mesh=mesh,
        scratch_types=[
            pltpu.VMEM((ch,), jnp.int32),
            pltpu.VMEM((ch,), jnp.int32),
            pltpu.VMEM((ch, D), jnp.float32),
            pltpu.VMEM((ch, D), jnp.float32),
            pltpu.SemaphoreType.DMA,
            pltpu.SemaphoreType.DMA,
        ],
    )
    def gather(table_hbm, idx_hbm, out_hbm, idx_a, idx_b, rows_a, rows_b,
               sem_a, sem_b):
        wid = lax.axis_index("s") * info.num_cores + lax.axis_index("c")
        base = wid * per_w
        idx_v = (idx_a, idx_b)
        rows_v = (rows_a, rows_b)
        sems = (sem_a, sem_b)

        def body(i, _):
            handles = []
            for b in range(2):
                off = base + (2 * i + b) * ch
                pltpu.sync_copy(idx_hbm.at[pl.ds(off, ch)], idx_v[b])
                handles.append(
                    pltpu.async_copy(table_hbm.at[idx_v[b]], rows_v[b],
                                     sems[b]))
            for b in range(2):
                off = base + (2 * i + b) * ch
                handles[b].wait()
                pltpu.sync_copy(rows_v[b], out_hbm.at[pl.ds(off, ch)])
            return ()

        lax.fori_loop(0, n_pairs, body, ())

    return gather


def kernel(f, x_e, src, Wq, bq, Wk, bk, Wv, bv, Wo, bo, W0, b0, W1, b1,
           Wl, bl):
    bq2, bk2, bv2, bo2, b02, b12, bl2 = (
        b.reshape(1, D) for b in (bq, bk, bv, bo, b0, b1, bl))

    p1 = pl.pallas_call(
        _p1_body,
        grid=(GRID,),
        in_specs=[_edge_spec(), _node_spec(),
                  _w_spec(D), _b_spec(), _w_spec(D), _b_spec(),
                  _w_spec(D), _b_spec(), _w_spec(D), _b_spec()],
        out_specs=_node_spec(),
        out_shape=jax.ShapeDtypeStruct((N, D), jnp.float32),
    )
    fh1 = p1(x_e, f, Wq, bq2, Wk, bk2, Wv, bv2, Wo, bo2)

    sc_gather = _make_sc_gather()
    g0 = sc_gather(fh1, src)

    p2 = pl.pallas_call(
        _p2_body,
        grid=(GRID,),
        in_specs=[_edge_spec(), _edge_spec(), _node_spec(),
                  _w_spec(D), _b_spec(), _w_spec(D), _b_spec(),
                  _w_spec(D), _b_spec(), _w_spec(D), _b_spec(),
                  _w_spec(D), _b_spec()],
        out_specs=[_edge_spec(), _node_spec()],
        out_shape=[jax.ShapeDtypeStruct((E, D), jnp.float32),
                   jax.ShapeDtypeStruct((N, D), jnp.float32)],
    )
    h1, fh2 = p2(x_e, g0, fh1, Wq, bq2, Wk, bk2, Wv, bv2, Wo, bo2, W0, b02)

    g1 = sc_gather(fh2, src)

    p3 = pl.pallas_call(
        _p3_body,
        grid=(GRID,),
        in_specs=[_edge_spec(), _edge_spec(), _edge_spec(), _node_spec(),
                  _node_spec(), _w_spec(D), _b_spec(),
                  pl.BlockSpec((3 * D, D), lambda i: (0, 0)), _b_spec()],
        out_specs=_node_spec(),
        out_shape=jax.ShapeDtypeStruct((N, D), jnp.float32),
    )
    return p3(x_e, g1, h1, fh2, f, W1, b12, Wl, bl2)
